# baseline (speedup 1.0000x reference)
import functools

import jax
import jax.numpy as jnp
from jax import lax
from jax.experimental import pallas as pl
from jax.experimental.pallas import tpu as pltpu
from jax.experimental.pallas import tpu_sc as plsc

N_NODES = 10000
EMB = 128
N_EDGES = 320000

NC = 2
NS = 16
LANES = 16

E_PAD = 327680
CHUNK = 128
NBUF = 2
EDGES_PER_TILE = E_PAD // NS
N_CHUNKS = EDGES_PER_TILE // CHUNK
ACC_ROWS = 10240
ZROWS_PER_TILE = ACC_ROWS // NS
DUMP_BASE = N_NODES


def _lrelu(v):
    return jnp.where(v >= 0, v, 0.01 * v)


CH_ROWS = E_PAD // CHUNK
CH_PER_TILE = CH_ROWS // NS
IDX_BLK = 16
N_BLKS = CH_PER_TILE // IDX_BLK


def _sc_agg_body(x_hbm, src2_hbm, sidx2_hbm, agg_out,
                 acc, srcs, sidxs, r0, r1, zrow,
                 sg0, sg1, ss0, ss1):
    rows = (r0, r1)
    semg = (sg0, sg1)
    sems = (ss0, ss1)
    rel = lax.axis_index("c")
    tid = lax.axis_index("s")

    z16 = jnp.zeros((LANES,), jnp.float32)
    for r in range(16):
        for c in range(EMB // LANES):
            zrow[r, pl.ds(LANES * c, LANES)] = z16

    def _z(k, carry):
        pltpu.sync_copy(zrow, acc.at[pl.ds(tid * ZROWS_PER_TILE + k * 16, 16)])
        return carry
    lax.fori_loop(0, ZROWS_PER_TILE // 16, _z, 0)
    plsc.subcore_barrier()

    H = CHUNK // 2

    def _gath(i, buf, sem):
        pltpu.async_copy(x_hbm.at[srcs.at[i, pl.ds(0, H)]],
                         buf.at[pl.ds(0, H)], sem)
        pltpu.async_copy(x_hbm.at[srcs.at[i, pl.ds(H, H)]],
                         buf.at[pl.ds(H, H)], sem)

    def _wait2(buf, sem):
        pltpu.make_async_copy(x_hbm.at[srcs.at[0, pl.ds(0, H)]],
                              buf.at[pl.ds(0, H)], sem).wait()
        pltpu.make_async_copy(x_hbm.at[srcs.at[0, pl.ds(0, H)]],
                              buf.at[pl.ds(H, H)], sem).wait()

    def _blk(b, carry):
        pltpu.sync_copy(
            src2_hbm.at[pl.ds(rel * CH_ROWS + tid * CH_PER_TILE + b * IDX_BLK,
                              IDX_BLK)], srcs)
        pltpu.sync_copy(
            sidx2_hbm.at[pl.ds(rel * CH_ROWS + tid * CH_PER_TILE + b * IDX_BLK,
                               IDX_BLK)], sidxs)
        _gath(0, rows[0], semg[0])

        def _pair(g, c2):
            i0 = 2 * g
            _wait2(rows[0], semg[0])
            _gath(i0 + 1, rows[1], semg[1])
            pltpu.sync_copy(rows[0], acc.at[sidxs.at[i0]], add=True)
            nxt = jnp.where(i0 + 2 < IDX_BLK, i0 + 2, 0)
            _wait2(rows[1], semg[1])
            _gath(nxt, rows[0], semg[0])
            pltpu.sync_copy(rows[1], acc.at[sidxs.at[i0 + 1]], add=True)
            return c2
        lax.fori_loop(0, IDX_BLK // 2, _pair, 0)
        _wait2(rows[0], semg[0])
        return carry
    lax.fori_loop(0, N_BLKS, _blk, 0)
    plsc.subcore_barrier()

    obase = tid * ZROWS_PER_TILE

    def _o(k, carry):
        pltpu.sync_copy(acc.at[pl.ds(obase + k * CHUNK, CHUNK)], r0)
        pltpu.sync_copy(r0, agg_out.at[rel].at[pl.ds(obase + k * CHUNK, CHUNK)])
        return carry
    lax.fori_loop(0, ZROWS_PER_TILE // CHUNK, _o, 0)


@jax.jit
def _sc_agg(x, src2, sidx2):
    f = pl.kernel(
        _sc_agg_body,
        out_type=[jax.ShapeDtypeStruct((NC, ACC_ROWS, EMB), jnp.float32)],
        mesh=plsc.VectorSubcoreMesh(core_axis_name="c", subcore_axis_name="s"),
        scratch_types=[
            pltpu.VMEM_SHARED((ACC_ROWS, EMB), jnp.float32),
            pltpu.VMEM((IDX_BLK, CHUNK), jnp.int32),
            pltpu.VMEM((IDX_BLK, CHUNK), jnp.int32),
            pltpu.VMEM((CHUNK, EMB), jnp.float32),
            pltpu.VMEM((CHUNK, EMB), jnp.float32),
            pltpu.VMEM((16, EMB), jnp.float32),
            pltpu.SemaphoreType.DMA,
            pltpu.SemaphoreType.DMA,
            pltpu.SemaphoreType.DMA,
            pltpu.SemaphoreType.DMA,
        ],
    )
    return f(x, src2, sidx2)[0]


def _sc_cnt_body(ones_hbm, sidx2_hbm, cnt_out, acc, sidxs, ones_v, zrow, sem):
    rel = lax.axis_index("c")
    tid = lax.axis_index("s")

    z16 = jnp.zeros((LANES,), jnp.float32)
    for r in range(16):
        for c in range(EMB // LANES):
            zrow[r, pl.ds(LANES * c, LANES)] = z16
    pltpu.sync_copy(ones_hbm, ones_v)

    def _z(k, carry):
        pltpu.sync_copy(zrow, acc.at[pl.ds(tid * ZROWS_PER_TILE + k * 16, 16)])
        return carry
    lax.fori_loop(0, ZROWS_PER_TILE // 16, _z, 0)
    plsc.subcore_barrier()

    def _blk(b, carry):
        pltpu.sync_copy(
            sidx2_hbm.at[pl.ds(rel * CH_ROWS + tid * CH_PER_TILE + b * IDX_BLK,
                               IDX_BLK)], sidxs)

        def _chunk(i, c2):
            pltpu.sync_copy(ones_v, acc.at[sidxs.at[i]], add=True)
            return c2
        lax.fori_loop(0, IDX_BLK, _chunk, 0)
        return carry
    lax.fori_loop(0, N_BLKS, _blk, 0)
    plsc.subcore_barrier()

    obase = tid * ZROWS_PER_TILE

    def _o(k, carry):
        pltpu.sync_copy(acc.at[pl.ds(obase + k * CHUNK, CHUNK)], ones_v)
        pltpu.sync_copy(ones_v, cnt_out.at[rel].at[pl.ds(obase + k * CHUNK, CHUNK)])
        return carry
    lax.fori_loop(0, ZROWS_PER_TILE // CHUNK, _o, 0)


@jax.jit
def _sc_cnt(ones_rows, sidx):
    f = pl.kernel(
        _sc_cnt_body,
        out_type=[jax.ShapeDtypeStruct((NC, ACC_ROWS, EMB), jnp.float32)],
        mesh=plsc.VectorSubcoreMesh(core_axis_name="c", subcore_axis_name="s"),
        scratch_types=[
            pltpu.VMEM_SHARED((ACC_ROWS, EMB), jnp.float32),
            pltpu.VMEM((IDX_BLK, CHUNK), jnp.int32),
            pltpu.VMEM((CHUNK, EMB), jnp.float32),
            pltpu.VMEM((16, EMB), jnp.float32),
            pltpu.SemaphoreType.DMA,
        ],
    )
    return f(ones_rows, sidx)[0]


N_PAD = ACC_ROWS
_BLK = 1280
_GRID = N_PAD // _BLK


def _enc_body(x_ref, wbd_ref, bbd_ref, win_ref, bin_ref, o_ref):
    h = _lrelu(jnp.dot(x_ref[...], wbd_ref[...],
                       preferred_element_type=jnp.float32) + bbd_ref[...])
    o_ref[...] = _lrelu(jnp.dot(h, win_ref[...],
                                preferred_element_type=jnp.float32) + bin_ref[...])


def _encoder(xcat, wbd, bbd, w_in, b_in):
    d_in = xcat.shape[1]
    return pl.pallas_call(
        _enc_body,
        grid=(_GRID,),
        in_specs=[
            pl.BlockSpec((_BLK, d_in), lambda i: (i, 0)),
            pl.BlockSpec((d_in, EMB), lambda i: (0, 0)),
            pl.BlockSpec((1, EMB), lambda i: (0, 0)),
            pl.BlockSpec((EMB, EMB), lambda i: (0, 0)),
            pl.BlockSpec((1, EMB), lambda i: (0, 0)),
        ],
        out_specs=pl.BlockSpec((_BLK, EMB), lambda i: (i, 0)),
        out_shape=jax.ShapeDtypeStruct((N_PAD, EMB), jnp.float32),
    )(xcat, wbd, bbd, w_in, b_in)


def _combine_core(x_ref, a0_ref, a1_ref, c0_ref, c1_ref,
                  wroot_ref, w0_ref, w1_ref, b_ref):
    out = jnp.dot(x_ref[...], wroot_ref[...],
                  preferred_element_type=jnp.float32) + b_ref[...]
    c0 = c0_ref[...][:, 0:1]
    c1 = c1_ref[...][:, 0:1]
    out = out + jnp.dot(a0_ref[...], w0_ref[...],
                        preferred_element_type=jnp.float32) / jnp.maximum(c0, 1.0)
    out = out + jnp.dot(a1_ref[...], w1_ref[...],
                        preferred_element_type=jnp.float32) / jnp.maximum(c1, 1.0)
    return out


def _comb_body(x_ref, a0_ref, a1_ref, c0_ref, c1_ref,
               wroot_ref, w0_ref, w1_ref, b_ref, o_ref):
    o_ref[...] = _combine_core(x_ref, a0_ref, a1_ref, c0_ref, c1_ref,
                               wroot_ref, w0_ref, w1_ref, b_ref)


def _comb_head_body(x_ref, a0_ref, a1_ref, c0_ref, c1_ref,
                    wroot_ref, w0_ref, w1_ref, b_ref,
                    wo1_ref, bo1_ref, wo2_ref, bo2_ref, o_ref):
    out = _combine_core(x_ref, a0_ref, a1_ref, c0_ref, c1_ref,
                        wroot_ref, w0_ref, w1_ref, b_ref)
    y = _lrelu(jnp.dot(out, wo1_ref[...],
                       preferred_element_type=jnp.float32) + bo1_ref[...])
    o_ref[...] = jnp.dot(y, wo2_ref[...],
                         preferred_element_type=jnp.float32) + bo2_ref[...]


def _x_specs():
    return [
        pl.BlockSpec((_BLK, EMB), lambda i: (i, 0)),
        pl.BlockSpec((_BLK, EMB), lambda i: (i, 0)),
        pl.BlockSpec((_BLK, EMB), lambda i: (i, 0)),
        pl.BlockSpec((_BLK, EMB), lambda i: (i, 0)),
        pl.BlockSpec((_BLK, EMB), lambda i: (i, 0)),
        pl.BlockSpec((EMB, EMB), lambda i: (0, 0)),
        pl.BlockSpec((EMB, EMB), lambda i: (0, 0)),
        pl.BlockSpec((EMB, EMB), lambda i: (0, 0)),
        pl.BlockSpec((1, EMB), lambda i: (0, 0)),
    ]


def _combine(x, a0, a1, c0, c1, wroot, w0, w1, b):
    return pl.pallas_call(
        _comb_body,
        grid=(_GRID,),
        in_specs=_x_specs(),
        out_specs=pl.BlockSpec((_BLK, EMB), lambda i: (i, 0)),
        out_shape=jax.ShapeDtypeStruct((N_PAD, EMB), jnp.float32),
    )(x, a0, a1, c0, c1, wroot, w0, w1, b)


def _combine_head(x, a0, a1, c0, c1, wroot, w0, w1, b,
                  wo1, bo1, wo2, bo2):
    return pl.pallas_call(
        _comb_head_body,
        grid=(_GRID,),
        in_specs=_x_specs() + [
            pl.BlockSpec((EMB, EMB), lambda i: (0, 0)),
            pl.BlockSpec((1, EMB), lambda i: (0, 0)),
            pl.BlockSpec((EMB, EMB), lambda i: (0, 0)),
            pl.BlockSpec((1, EMB), lambda i: (0, 0)),
        ],
        out_specs=pl.BlockSpec((_BLK, EMB), lambda i: (i, 0)),
        out_shape=jax.ShapeDtypeStruct((N_PAD, EMB), jnp.float32),
    )(x, a0, a1, c0, c1, wroot, w0, w1, b, wo1, bo1, wo2, bo2)


def kernel(des, tweet, num_prop, cat_prop, edge_index, edge_type,
           W_des, b_des, W_tweet, b_tweet, W_num, b_num, W_cat, b_cat,
           W_in, b_in, W_rel, W_root, b_rgcn, W_out1, b_out1, W_out2, b_out2):
    src = edge_index[0].astype(jnp.int32)
    dst = edge_index[1].astype(jnp.int32)
    et = edge_type.astype(jnp.int32)

    src = jnp.pad(src, (0, E_PAD - N_EDGES))
    dst = jnp.pad(dst, (0, E_PAD - N_EDGES))
    et = jnp.pad(et, (0, E_PAD - N_EDGES), constant_values=2)
    eid = jnp.arange(E_PAD, dtype=jnp.int32)
    dump = DUMP_BASE + (eid & 15)
    sidx = jnp.concatenate([jnp.where(et == 0, dst, dump),
                            jnp.where(et == 1, dst, dump)])
    ones_rows = jnp.ones((CHUNK, EMB), jnp.float32)
    sidx = sidx.reshape(NC * CH_ROWS, CHUNK)
    src2 = jnp.concatenate([jnp.where(et == 0, src, 0),
                            jnp.where(et == 1, src, 0)]).reshape(NC * CH_ROWS, CHUNK)

    xcat = jnp.concatenate([des, tweet, num_prop, cat_prop], axis=1)
    xcat = jnp.pad(xcat, ((0, N_PAD - N_NODES), (0, 0)))
    d_in = xcat.shape[1]
    q = EMB // 4
    wbd = jnp.zeros((d_in, EMB), jnp.float32)
    o = 0
    for w, col in ((W_des, 0), (W_tweet, 1), (W_num, 2), (W_cat, 3)):
        wbd = wbd.at[o:o + w.shape[0], col * q:(col + 1) * q].set(w)
        o += w.shape[0]
    bbd = jnp.concatenate([b_des, b_tweet, b_num, b_cat]).reshape(1, EMB)

    x = _encoder(xcat, wbd, bbd, W_in, b_in.reshape(1, EMB))

    cnt = _sc_cnt(ones_rows, sidx)
    w0, w1 = W_rel[0], W_rel[1]
    b = b_rgcn.reshape(1, EMB)

    agg = _sc_agg(x, src2, sidx)
    x = _combine(x, agg[0], agg[1], cnt[0], cnt[1], W_root, w0, w1, b)

    agg = _sc_agg(x, src2, sidx)
    wo2 = jnp.zeros((EMB, EMB), jnp.float32).at[:, :2].set(W_out2)
    bo2 = jnp.zeros((1, EMB), jnp.float32).at[0, :2].set(b_out2)
    out = _combine_head(x, agg[0], agg[1], cnt[0], cnt[1], W_root, w0, w1, b,
                        W_out1, b_out1.reshape(1, EMB), wo2, bo2)
    return out[:N_NODES, :2]


# speedup vs baseline: 14.3365x; 14.3365x over previous
import functools

import jax
import jax.numpy as jnp
from jax import lax
from jax.experimental import pallas as pl
from jax.experimental.pallas import tpu as pltpu
from jax.experimental.pallas import tpu_sc as plsc

N_NODES = 10000
EMB = 128
N_EDGES = 320000

NC = 2
NS = 16
LANES = 16

E_PAD = 327680
CHUNK = 128
NBUF = 2
EDGES_PER_TILE = E_PAD // NS
N_CHUNKS = EDGES_PER_TILE // CHUNK
ACC_ROWS = 10240
ZROWS_PER_TILE = ACC_ROWS // NS
DUMP_BASE = N_NODES


def _lrelu(v):
    return jnp.where(v >= 0, v, 0.01 * v)


CH_ROWS = E_PAD // CHUNK
CH_PER_TILE = CH_ROWS // NS
IDX_BLK = 16
N_BLKS = CH_PER_TILE // IDX_BLK


def _sc_agg_body(x_hbm, src2_hbm, sidx2_hbm, agg_out,
                 acc, srcs, sidxs, r0, r1, zrow,
                 sg0, sg1, ss0, ss1):
    rows = (r0, r1)
    semg = (sg0, sg1)
    sems = (ss0, ss1)
    rel = lax.axis_index("c")
    tid = lax.axis_index("s")

    z16 = jnp.zeros((LANES,), jnp.float32)
    for r in range(16):
        for c in range(EMB // LANES):
            zrow[r, pl.ds(LANES * c, LANES)] = z16

    def _z(k, carry):
        pltpu.sync_copy(zrow, acc.at[pl.ds(tid * ZROWS_PER_TILE + k * 16, 16)])
        return carry
    lax.fori_loop(0, ZROWS_PER_TILE // 16, _z, 0)
    plsc.subcore_barrier()

    H = CHUNK // 2

    def _gath(i, buf, sem):
        pltpu.async_copy(x_hbm.at[srcs.at[i, pl.ds(0, H)]],
                         buf.at[pl.ds(0, H)], sem)
        pltpu.async_copy(x_hbm.at[srcs.at[i, pl.ds(H, H)]],
                         buf.at[pl.ds(H, H)], sem)

    def _wait2(buf, sem):
        pltpu.make_async_copy(x_hbm.at[srcs.at[0, pl.ds(0, H)]],
                              buf.at[pl.ds(0, H)], sem).wait()
        pltpu.make_async_copy(x_hbm.at[srcs.at[0, pl.ds(0, H)]],
                              buf.at[pl.ds(H, H)], sem).wait()

    def _blk(b, carry):
        pltpu.sync_copy(
            src2_hbm.at[pl.ds(tid * CH_PER_TILE + b * IDX_BLK, IDX_BLK)], srcs)
        pltpu.sync_copy(
            sidx2_hbm.at[pl.ds(rel * CH_ROWS + tid * CH_PER_TILE + b * IDX_BLK,
                               IDX_BLK)], sidxs)
        _gath(0, rows[0], semg[0])

        def _pair(g, c2):
            i0 = 2 * g
            _wait2(rows[0], semg[0])
            _gath(i0 + 1, rows[1], semg[1])
            pltpu.sync_copy(rows[0], acc.at[sidxs.at[i0]], add=True)
            nxt = jnp.where(i0 + 2 < IDX_BLK, i0 + 2, 0)
            _wait2(rows[1], semg[1])
            _gath(nxt, rows[0], semg[0])
            pltpu.sync_copy(rows[1], acc.at[sidxs.at[i0 + 1]], add=True)
            return c2
        lax.fori_loop(0, IDX_BLK // 2, _pair, 0)
        _wait2(rows[0], semg[0])
        return carry
    lax.fori_loop(0, N_BLKS, _blk, 0)
    plsc.subcore_barrier()

    obase = tid * ZROWS_PER_TILE

    def _o(k, carry):
        pltpu.sync_copy(acc.at[pl.ds(obase + k * CHUNK, CHUNK)], r0)
        pltpu.sync_copy(r0, agg_out.at[rel].at[pl.ds(obase + k * CHUNK, CHUNK)])
        return carry
    lax.fori_loop(0, ZROWS_PER_TILE // CHUNK, _o, 0)


@jax.jit
def _sc_agg(x, src2, sidx2):
    f = pl.kernel(
        _sc_agg_body,
        out_type=[jax.ShapeDtypeStruct((NC, ACC_ROWS, EMB), jnp.float32)],
        mesh=plsc.VectorSubcoreMesh(core_axis_name="c", subcore_axis_name="s"),
        scratch_types=[
            pltpu.VMEM_SHARED((ACC_ROWS, EMB), jnp.float32),
            pltpu.VMEM((IDX_BLK, CHUNK), jnp.int32),
            pltpu.VMEM((IDX_BLK, CHUNK), jnp.int32),
            pltpu.VMEM((CHUNK, EMB), jnp.float32),
            pltpu.VMEM((CHUNK, EMB), jnp.float32),
            pltpu.VMEM((16, EMB), jnp.float32),
            pltpu.SemaphoreType.DMA,
            pltpu.SemaphoreType.DMA,
            pltpu.SemaphoreType.DMA,
            pltpu.SemaphoreType.DMA,
        ],
    )
    return f(x, src2, sidx2)[0]


def _sc_cnt_body(ones_hbm, sidx2_hbm, cnt_out, acc, sidxs, ones_v, zrow, sem):
    rel = lax.axis_index("c")
    tid = lax.axis_index("s")

    z16 = jnp.zeros((LANES,), jnp.float32)
    for r in range(16):
        for c in range(EMB // LANES):
            zrow[r, pl.ds(LANES * c, LANES)] = z16
    pltpu.sync_copy(ones_hbm, ones_v)

    def _z(k, carry):
        pltpu.sync_copy(zrow, acc.at[pl.ds(tid * ZROWS_PER_TILE + k * 16, 16)])
        return carry
    lax.fori_loop(0, ZROWS_PER_TILE // 16, _z, 0)
    plsc.subcore_barrier()

    def _blk(b, carry):
        pltpu.sync_copy(
            sidx2_hbm.at[pl.ds(rel * CH_ROWS + tid * CH_PER_TILE + b * IDX_BLK,
                               IDX_BLK)], sidxs)

        def _chunk(i, c2):
            pltpu.sync_copy(ones_v, acc.at[sidxs.at[i]], add=True)
            return c2
        lax.fori_loop(0, IDX_BLK, _chunk, 0)
        return carry
    lax.fori_loop(0, N_BLKS, _blk, 0)
    plsc.subcore_barrier()

    obase = tid * ZROWS_PER_TILE

    def _o(k, carry):
        pltpu.sync_copy(acc.at[pl.ds(obase + k * CHUNK, CHUNK)], ones_v)
        pltpu.sync_copy(ones_v, cnt_out.at[rel].at[pl.ds(obase + k * CHUNK, CHUNK)])
        return carry
    lax.fori_loop(0, ZROWS_PER_TILE // CHUNK, _o, 0)


@jax.jit
def _sc_cnt(ones_rows, sidx):
    f = pl.kernel(
        _sc_cnt_body,
        out_type=[jax.ShapeDtypeStruct((NC, ACC_ROWS, EMB), jnp.float32)],
        mesh=plsc.VectorSubcoreMesh(core_axis_name="c", subcore_axis_name="s"),
        scratch_types=[
            pltpu.VMEM_SHARED((ACC_ROWS, EMB), jnp.float32),
            pltpu.VMEM((IDX_BLK, CHUNK), jnp.int32),
            pltpu.VMEM((CHUNK, EMB), jnp.float32),
            pltpu.VMEM((16, EMB), jnp.float32),
            pltpu.SemaphoreType.DMA,
        ],
    )
    return f(ones_rows, sidx)[0]


N_PAD = ACC_ROWS
_BLK = 1280
_GRID = N_PAD // _BLK


def _enc_body(x_ref, wbd_ref, bbd_ref, win_ref, bin_ref, o_ref):
    h = _lrelu(jnp.dot(x_ref[...], wbd_ref[...],
                       preferred_element_type=jnp.float32) + bbd_ref[...])
    o_ref[...] = _lrelu(jnp.dot(h, win_ref[...],
                                preferred_element_type=jnp.float32) + bin_ref[...])


def _encoder(xcat, wbd, bbd, w_in, b_in):
    d_in = xcat.shape[1]
    return pl.pallas_call(
        _enc_body,
        grid=(_GRID,),
        in_specs=[
            pl.BlockSpec((_BLK, d_in), lambda i: (i, 0)),
            pl.BlockSpec((d_in, EMB), lambda i: (0, 0)),
            pl.BlockSpec((1, EMB), lambda i: (0, 0)),
            pl.BlockSpec((EMB, EMB), lambda i: (0, 0)),
            pl.BlockSpec((1, EMB), lambda i: (0, 0)),
        ],
        out_specs=pl.BlockSpec((_BLK, EMB), lambda i: (i, 0)),
        out_shape=jax.ShapeDtypeStruct((N_PAD, EMB), jnp.float32),
    )(xcat, wbd, bbd, w_in, b_in)


def _combine_core(x_ref, a0_ref, a1_ref, c0_ref, c1_ref,
                  wroot_ref, w0_ref, w1_ref, b_ref):
    out = jnp.dot(x_ref[...], wroot_ref[...],
                  preferred_element_type=jnp.float32) + b_ref[...]
    c0 = c0_ref[...][:, 0:1]
    c1 = c1_ref[...][:, 0:1]
    out = out + jnp.dot(a0_ref[...], w0_ref[...],
                        preferred_element_type=jnp.float32) / jnp.maximum(c0, 1.0)
    out = out + jnp.dot(a1_ref[...], w1_ref[...],
                        preferred_element_type=jnp.float32) / jnp.maximum(c1, 1.0)
    return out


def _comb_body(x_ref, a0_ref, a1_ref, c0_ref, c1_ref,
               wroot_ref, w0_ref, w1_ref, b_ref, o_ref):
    o_ref[...] = _combine_core(x_ref, a0_ref, a1_ref, c0_ref, c1_ref,
                               wroot_ref, w0_ref, w1_ref, b_ref)


def _comb_head_body(x_ref, a0_ref, a1_ref, c0_ref, c1_ref,
                    wroot_ref, w0_ref, w1_ref, b_ref,
                    wo1_ref, bo1_ref, wo2_ref, bo2_ref, o_ref):
    out = _combine_core(x_ref, a0_ref, a1_ref, c0_ref, c1_ref,
                        wroot_ref, w0_ref, w1_ref, b_ref)
    y = _lrelu(jnp.dot(out, wo1_ref[...],
                       preferred_element_type=jnp.float32) + bo1_ref[...])
    o_ref[...] = jnp.dot(y, wo2_ref[...],
                         preferred_element_type=jnp.float32) + bo2_ref[...]


def _x_specs():
    return [
        pl.BlockSpec((_BLK, EMB), lambda i: (i, 0)),
        pl.BlockSpec((_BLK, EMB), lambda i: (i, 0)),
        pl.BlockSpec((_BLK, EMB), lambda i: (i, 0)),
        pl.BlockSpec((_BLK, EMB), lambda i: (i, 0)),
        pl.BlockSpec((_BLK, EMB), lambda i: (i, 0)),
        pl.BlockSpec((EMB, EMB), lambda i: (0, 0)),
        pl.BlockSpec((EMB, EMB), lambda i: (0, 0)),
        pl.BlockSpec((EMB, EMB), lambda i: (0, 0)),
        pl.BlockSpec((1, EMB), lambda i: (0, 0)),
    ]


def _combine(x, a0, a1, c0, c1, wroot, w0, w1, b):
    return pl.pallas_call(
        _comb_body,
        grid=(_GRID,),
        in_specs=_x_specs(),
        out_specs=pl.BlockSpec((_BLK, EMB), lambda i: (i, 0)),
        out_shape=jax.ShapeDtypeStruct((N_PAD, EMB), jnp.float32),
    )(x, a0, a1, c0, c1, wroot, w0, w1, b)


def _combine_head(x, a0, a1, c0, c1, wroot, w0, w1, b,
                  wo1, bo1, wo2, bo2):
    return pl.pallas_call(
        _comb_head_body,
        grid=(_GRID,),
        in_specs=_x_specs() + [
            pl.BlockSpec((EMB, EMB), lambda i: (0, 0)),
            pl.BlockSpec((1, EMB), lambda i: (0, 0)),
            pl.BlockSpec((EMB, EMB), lambda i: (0, 0)),
            pl.BlockSpec((1, EMB), lambda i: (0, 0)),
        ],
        out_specs=pl.BlockSpec((_BLK, EMB), lambda i: (i, 0)),
        out_shape=jax.ShapeDtypeStruct((N_PAD, EMB), jnp.float32),
    )(x, a0, a1, c0, c1, wroot, w0, w1, b, wo1, bo1, wo2, bo2)


def kernel(des, tweet, num_prop, cat_prop, edge_index, edge_type,
           W_des, b_des, W_tweet, b_tweet, W_num, b_num, W_cat, b_cat,
           W_in, b_in, W_rel, W_root, b_rgcn, W_out1, b_out1, W_out2, b_out2):
    src = edge_index[0].astype(jnp.int32)
    dst = edge_index[1].astype(jnp.int32)
    et = edge_type.astype(jnp.int32)

    src = jnp.pad(src, (0, E_PAD - N_EDGES))
    dst = jnp.pad(dst, (0, E_PAD - N_EDGES))
    et = jnp.pad(et, (0, E_PAD - N_EDGES), constant_values=2)
    eid = jnp.arange(E_PAD, dtype=jnp.int32)
    dump = DUMP_BASE + (eid & 15)
    sidx = jnp.concatenate([jnp.where(et == 0, dst, dump),
                            jnp.where(et == 1, dst, dump)])
    ones_rows = jnp.ones((CHUNK, EMB), jnp.float32)
    sidx = sidx.reshape(NC * CH_ROWS, CHUNK)
    src2 = src.reshape(CH_ROWS, CHUNK)

    xcat = jnp.concatenate([des, tweet, num_prop, cat_prop], axis=1)
    xcat = jnp.pad(xcat, ((0, N_PAD - N_NODES), (0, 0)))
    d_in = xcat.shape[1]
    q = EMB // 4
    wbd = jnp.zeros((d_in, EMB), jnp.float32)
    o = 0
    for w, col in ((W_des, 0), (W_tweet, 1), (W_num, 2), (W_cat, 3)):
        wbd = wbd.at[o:o + w.shape[0], col * q:(col + 1) * q].set(w)
        o += w.shape[0]
    bbd = jnp.concatenate([b_des, b_tweet, b_num, b_cat]).reshape(1, EMB)

    x = _encoder(xcat, wbd, bbd, W_in, b_in.reshape(1, EMB))

    cnt = _sc_cnt(ones_rows, sidx)
    w0, w1 = W_rel[0], W_rel[1]
    b = b_rgcn.reshape(1, EMB)

    agg = _sc_agg(x, src2, sidx)
    x = _combine(x, agg[0], agg[1], cnt[0], cnt[1], W_root, w0, w1, b)

    agg = _sc_agg(x, src2, sidx)
    wo2 = jnp.zeros((EMB, EMB), jnp.float32).at[:, :2].set(W_out2)
    bo2 = jnp.zeros((1, EMB), jnp.float32).at[0, :2].set(b_out2)
    out = _combine_head(x, agg[0], agg[1], cnt[0], cnt[1], W_root, w0, w1, b,
                        W_out1, b_out1.reshape(1, EMB), wo2, bo2)
    return out[:N_NODES, :2]
